# baseline (speedup 1.0000x reference)
import functools

import jax
import jax.numpy as jnp
from jax import lax
from jax.experimental import pallas as pl
from jax.experimental.pallas import tpu as pltpu
from jax.experimental.pallas import tpu_sc as plsc

ALPHA = 0.2
N = 10000
NPAD = 10240
D = 256
DP = D // 2
M = 3
S = 16
NCLASS = 64
TT = M * NPAD
NWORK = 32
TPW = TT // NWORK
NPW = NPAD // NWORK
BLK = 8
NPAIR = NPW // BLK // 2
NSTG = 10048
STG = 632
TB = 512


def _pack_rows(h):
    hb = h.astype(jnp.bfloat16)
    hu = jax.lax.bitcast_convert_type(hb, jnp.uint16).astype(jnp.uint32)
    word = hu[:, :DP] | (hu[:, DP:] << 16)
    return jax.lax.bitcast_convert_type(word, jnp.float32)


def _unpack_rows(hp):
    w = jax.lax.bitcast_convert_type(hp, jnp.uint32)
    lo = jax.lax.bitcast_convert_type(w << 16, jnp.float32)
    hi = jax.lax.bitcast_convert_type(w & jnp.uint32(0xFFFF0000),
                                      jnp.float32)
    return jnp.concatenate([lo, hi], axis=-1)


def _proj_body(x_ref, w_ref, a_ref, h_ref, hp_ref, p_ref, q_ref):
    x = x_ref[...]
    h = jnp.dot(x, w_ref[0], preferred_element_type=jnp.float32)
    h_ref[0] = h
    hp_ref[0] = _pack_rows(h)
    a = a_ref[0]
    p_ref[0] = jnp.dot(h, a[:D], preferred_element_type=jnp.float32)
    q_ref[0] = jnp.dot(h, a[D:], preferred_element_type=jnp.float32)


def _proj(x, w, a, nfeat):
    return pl.pallas_call(
        _proj_body,
        grid=(M, NPAD // TB),
        in_specs=[
            pl.BlockSpec((TB, nfeat), lambda m, n: (n, 0)),
            pl.BlockSpec((1, nfeat, D), lambda m, n: (m, 0, 0)),
            pl.BlockSpec((1, 2 * D, 1), lambda m, n: (m, 0, 0)),
        ],
        out_specs=[
            pl.BlockSpec((1, TB, D), lambda m, n: (m, n, 0)),
            pl.BlockSpec((1, TB, DP), lambda m, n: (m, n, 0)),
            pl.BlockSpec((1, TB, 1), lambda m, n: (m, n, 0)),
            pl.BlockSpec((1, TB, 1), lambda m, n: (m, n, 0)),
        ],
        out_shape=[
            jax.ShapeDtypeStruct((M, NPAD, D), jnp.float32),
            jax.ShapeDtypeStruct((M, NPAD, DP), jnp.float32),
            jax.ShapeDtypeStruct((M, NPAD, 1), jnp.float32),
            jax.ShapeDtypeStruct((M, NPAD, 1), jnp.float32),
        ],
    )(x, w, a)


def _proj2_body(h_ref, g_ref, wa_ref, wb_ref, a_ref, h2_ref, hp_ref, p_ref,
                q_ref):
    rh = jnp.maximum(h_ref[0], 0.0)
    rg = jnp.maximum(_unpack_rows(g_ref[0]), 0.0)
    h2 = (jnp.dot(rh, wa_ref[0], preferred_element_type=jnp.float32)
          + jnp.dot(rg, wb_ref[0], preferred_element_type=jnp.float32))
    h2_ref[0] = h2
    hp_ref[0] = _pack_rows(h2)
    a = a_ref[0]
    p_ref[0] = jnp.dot(h2, a[:D], preferred_element_type=jnp.float32)
    q_ref[0] = jnp.dot(h2, a[D:], preferred_element_type=jnp.float32)


def _proj2(h, g, wa, wb, a):
    return pl.pallas_call(
        _proj2_body,
        grid=(M, NPAD // TB),
        in_specs=[
            pl.BlockSpec((1, TB, D), lambda m, n: (m, n, 0)),
            pl.BlockSpec((1, TB, DP), lambda m, n: (m, n, 0)),
            pl.BlockSpec((1, D, D), lambda m, n: (m, 0, 0)),
            pl.BlockSpec((1, D, D), lambda m, n: (m, 0, 0)),
            pl.BlockSpec((1, 2 * D, 1), lambda m, n: (m, 0, 0)),
        ],
        out_specs=[
            pl.BlockSpec((1, TB, D), lambda m, n: (m, n, 0)),
            pl.BlockSpec((1, TB, DP), lambda m, n: (m, n, 0)),
            pl.BlockSpec((1, TB, 1), lambda m, n: (m, n, 0)),
            pl.BlockSpec((1, TB, 1), lambda m, n: (m, n, 0)),
        ],
        out_shape=[
            jax.ShapeDtypeStruct((M, NPAD, D), jnp.float32),
            jax.ShapeDtypeStruct((M, NPAD, DP), jnp.float32),
            jax.ShapeDtypeStruct((M, NPAD, 1), jnp.float32),
            jax.ShapeDtypeStruct((M, NPAD, 1), jnp.float32),
        ],
    )(h, g, wa, wb, a)


_sc_mesh = plsc.VectorSubcoreMesh(core_axis_name="c", subcore_axis_name="s")


@functools.partial(
    pl.kernel,
    out_type=jax.ShapeDtypeStruct((TT, DP), jnp.float32),
    mesh=_sc_mesh,
    compiler_params=pltpu.CompilerParams(needs_layout_passes=False),
    scratch_types=[
        pltpu.VMEM_SHARED((NSTG, DP), jnp.float32),
        pltpu.VMEM((NSTG,), jnp.float32),
        pltpu.VMEM((NPW + S,), jnp.float32),
        pltpu.VMEM((NPW * S,), jnp.int32),
        pltpu.VMEM((BLK * S, DP), jnp.float32),
        pltpu.VMEM((BLK * S, DP), jnp.float32),
        pltpu.VMEM((2 * BLK, DP), jnp.float32),
        pltpu.SemaphoreType.DMA,
        pltpu.SemaphoreType.DMA,
        pltpu.SemaphoreType.DMA,
    ],
)
def _sc_agg(h_hbm, nbr_hbm, p_hbm, q_hbm, out_hbm,
            shared, q_v, p_v, nbr_v, rows0_v, rows1_v, ob_v,
            sem0, sem1, semo):
    cid = lax.axis_index("c")
    sid = lax.axis_index("s")
    wid = sid * 2 + cid
    def gather(blk, rows_v, sem):
        idx_ref = nbr_v.at[pl.ds(blk * (BLK * S), BLK * S)]
        pltpu.async_copy(shared.at[idx_ref], rows_v, sem)

    def wait_rows(rows_v, sem):
        pltpu.make_async_copy(
            shared.at[nbr_v.at[pl.ds(0, BLK * S)]], rows_v, sem).wait()

    def compute(blk, rows_v, half):
        for i in range(BLK):
            t = blk * BLK + i
            idx = nbr_v[pl.ds(t * S, S)]
            qn = plsc.load_gather(q_v, [idx])
            pv = p_v[pl.ds(t, S)][0]
            z = qn + pv
            e = jnp.maximum(z, ALPHA * z)
            ex = jnp.exp(e - jnp.max(e))
            att = ex / jnp.sum(ex)
            w = [att[s] for s in range(S)]
            himask = jnp.full((16,), -65536, jnp.int32)
            rnd = jnp.full((16,), 0x8000, jnp.int32)
            for dblk in range(DP // 16):
                sl = pl.ds(dblk * 16, 16)
                acc_a = None
                acc_b = None
                for s in range(S):
                    iw = plsc.bitcast(rows_v[i * S + s, sl], jnp.int32)
                    va = plsc.bitcast(iw << 16, jnp.float32)
                    vb = plsc.bitcast(iw & himask, jnp.float32)
                    if acc_a is None:
                        acc_a = w[s] * va
                        acc_b = w[s] * vb
                    else:
                        acc_a = acc_a + w[s] * va
                        acc_b = acc_b + w[s] * vb
                ia = plsc.bitcast(acc_a, jnp.int32)
                ib = plsc.bitcast(acc_b, jnp.int32)
                lo16 = lax.shift_right_logical(ia + rnd, 16)
                hi16 = (ib + rnd) & himask
                ob_v[half * BLK + i, sl] = plsc.bitcast(lo16 | hi16,
                                                        jnp.float32)

    def phase(m, carry):
        tb = m * NPAD
        pbase = wid * NPW
        @pl.when(sid < 15)
        def _stage_full():
            pltpu.sync_copy(h_hbm.at[pl.ds(tb + sid * STG, STG)],
                            shared.at[pl.ds(sid * STG, STG)])

        @pl.when(sid == 15)
        def _stage_tail():
            pltpu.sync_copy(h_hbm.at[pl.ds(tb + 15 * STG, NSTG - 15 * STG)],
                            shared.at[pl.ds(15 * STG, NSTG - 15 * STG)])
        pltpu.sync_copy(q_hbm.at[pl.ds(tb, NSTG)], q_v)
        pltpu.sync_copy(p_hbm.at[pl.ds(tb + pbase, NPW)],
                        p_v.at[pl.ds(0, NPW)])
        pltpu.sync_copy(nbr_hbm.at[pl.ds((tb + pbase) * S, NPW * S)], nbr_v)
        plsc.subcore_barrier()

        obase = tb + pbase
        gather(0, rows0_v, sem0)

        def pair(j, _):
            b0 = 2 * j
            gather(b0 + 1, rows1_v, sem1)
            @pl.when(j > 0)
            def _drain():
                pltpu.make_async_copy(
                    ob_v, out_hbm.at[pl.ds(obase, 2 * BLK)], semo).wait()
            wait_rows(rows0_v, sem0)
            compute(b0, rows0_v, 0)

            @pl.when(j < NPAIR - 1)
            def _prefetch():
                gather(b0 + 2, rows0_v, sem0)
            wait_rows(rows1_v, sem1)
            compute(b0 + 1, rows1_v, 1)
            pltpu.async_copy(
                ob_v, out_hbm.at[pl.ds(obase + b0 * BLK, 2 * BLK)], semo)
            return _

        lax.fori_loop(0, NPAIR, pair, None)
        pltpu.make_async_copy(
            ob_v, out_hbm.at[pl.ds(obase, 2 * BLK)], semo).wait()
        plsc.subcore_barrier()
        return carry

    lax.fori_loop(0, M, phase, None)


def _final_body(h_ref, g_ref, amp_ref, wl_ref, b_ref, o_ref):
    amp = amp_ref[...]
    a_top, a_bot = amp[:D], amp[D:]
    hs, gs, es = [], [], []
    for m in range(M):
        hm = jnp.maximum(h_ref[m], 0.0)
        gm = jnp.maximum(_unpack_rows(g_ref[m]), 0.0)
        e = (jnp.dot(hm, a_top, preferred_element_type=jnp.float32)
             + jnp.dot(gm, a_bot, preferred_element_type=jnp.float32))
        e = jnp.maximum(e, ALPHA * e)
        hs.append(hm)
        gs.append(gm)
        es.append(e)
    mx = jnp.maximum(jnp.maximum(es[0], es[1]), es[2])
    ws = [jnp.exp(e - mx) for e in es]
    tot = ws[0] + ws[1] + ws[2]
    ph = (ws[0] * hs[0] + ws[1] * hs[1] + ws[2] * hs[2]) / tot
    pg = (ws[0] * gs[0] + ws[1] * gs[1] + ws[2] * gs[2]) / tot
    wl = wl_ref[...]
    logits = (jnp.dot(ph, wl[:D], preferred_element_type=jnp.float32)
              + jnp.dot(pg, wl[D:], preferred_element_type=jnp.float32)
              + b_ref[...])
    r = jnp.maximum(logits, 0.0)
    rmx = jnp.max(r, axis=1, keepdims=True)
    lse = jnp.log(jnp.sum(jnp.exp(r - rmx), axis=1, keepdims=True)) + rmx
    o_ref[...] = r - lse


def _final(h2, g2, a_mp, w_lin, b_lin):
    return pl.pallas_call(
        _final_body,
        grid=(NPAD // TB,),
        in_specs=[
            pl.BlockSpec((M, TB, D), lambda n: (0, n, 0)),
            pl.BlockSpec((M, TB, DP), lambda n: (0, n, 0)),
            pl.BlockSpec((2 * D, 1), lambda n: (0, 0)),
            pl.BlockSpec((2 * D, NCLASS), lambda n: (0, 0)),
            pl.BlockSpec((NCLASS,), lambda n: (0,)),
        ],
        out_specs=pl.BlockSpec((TB, NCLASS), lambda n: (n, 0)),
        out_shape=jax.ShapeDtypeStruct((NPAD, NCLASS), jnp.float32),
    )(h2, g2, a_mp, w_lin, b_lin)


def _flat_nbr(nbr):
    nf = jnp.pad(nbr, ((0, 0), (0, NPAD - N), (0, 0)))
    return nf.reshape(TT * S)


def kernel(x, nbr1, nbr2, W1, a1, W2, a2, a_mp, W_lin, b_lin):
    xp = jnp.pad(x, ((0, NPAD - N), (0, 0)))
    nbr1f = _flat_nbr(nbr1)
    nbr2f = _flat_nbr(nbr2)

    h1, h1p, p1, q1 = _proj(xp, W1, a1, 512)
    agg1 = _sc_agg(h1p.reshape(TT, DP), nbr1f,
                   p1.reshape(TT), q1.reshape(TT))
    h2, h2p, p2, q2 = _proj2(h1, agg1.reshape(M, NPAD, DP),
                             W2[:, :D, :], W2[:, D:, :], a2)
    agg2 = _sc_agg(h2p.reshape(TT, DP), nbr2f,
                   p2.reshape(TT), q2.reshape(TT))
    out = _final(h2, agg2.reshape(M, NPAD, DP), a_mp, W_lin, b_lin)
    return out[:N]


# speedup vs baseline: 1.1461x; 1.1461x over previous
import functools

import jax
import jax.numpy as jnp
from jax import lax
from jax.experimental import pallas as pl
from jax.experimental.pallas import tpu as pltpu
from jax.experimental.pallas import tpu_sc as plsc

ALPHA = 0.2
N = 10000
NPAD = 10240
D = 256
DP = D // 2
M = 3
S = 16
NCLASS = 64
TT = M * NPAD
NWORK = 32
TPW = TT // NWORK
NPW = NPAD // NWORK
BLK = 4
NPAIR = NPW // BLK // 2
NSTG = 10112
STG = NSTG // 16
TB = 512


def _pack_rows(h):
    hb = h.astype(jnp.bfloat16)
    hu = jax.lax.bitcast_convert_type(hb, jnp.uint16).astype(jnp.uint32)
    word = hu[:, :DP] | (hu[:, DP:] << 16)
    return jax.lax.bitcast_convert_type(word, jnp.float32)


def _unpack_rows(hp):
    w = jax.lax.bitcast_convert_type(hp, jnp.uint32)
    lo = jax.lax.bitcast_convert_type(w << 16, jnp.float32)
    hi = jax.lax.bitcast_convert_type(w & jnp.uint32(0xFFFF0000),
                                      jnp.float32)
    return jnp.concatenate([lo, hi], axis=-1)


def _proj_body(x_ref, w_ref, a_ref, h_ref, hp_ref, p_ref, q_ref):
    x = x_ref[...]
    h = jnp.dot(x, w_ref[0], preferred_element_type=jnp.float32)
    h_ref[0] = h
    hp_ref[0] = _pack_rows(h)
    a = a_ref[0]
    p_ref[0] = jnp.dot(h, a[:D], preferred_element_type=jnp.float32)
    q_ref[0] = jnp.dot(h, a[D:], preferred_element_type=jnp.float32)


def _proj(x, w, a, nfeat):
    return pl.pallas_call(
        _proj_body,
        grid=(M, NPAD // TB),
        in_specs=[
            pl.BlockSpec((TB, nfeat), lambda m, n: (n, 0)),
            pl.BlockSpec((1, nfeat, D), lambda m, n: (m, 0, 0)),
            pl.BlockSpec((1, 2 * D, 1), lambda m, n: (m, 0, 0)),
        ],
        out_specs=[
            pl.BlockSpec((1, TB, D), lambda m, n: (m, n, 0)),
            pl.BlockSpec((1, TB, DP), lambda m, n: (m, n, 0)),
            pl.BlockSpec((1, TB, 1), lambda m, n: (m, n, 0)),
            pl.BlockSpec((1, TB, 1), lambda m, n: (m, n, 0)),
        ],
        out_shape=[
            jax.ShapeDtypeStruct((M, NPAD, D), jnp.float32),
            jax.ShapeDtypeStruct((M, NPAD, DP), jnp.float32),
            jax.ShapeDtypeStruct((M, NPAD, 1), jnp.float32),
            jax.ShapeDtypeStruct((M, NPAD, 1), jnp.float32),
        ],
    )(x, w, a)


def _proj2_body(h_ref, g_ref, wa_ref, wb_ref, a_ref, h2_ref, hp_ref, p_ref,
                q_ref):
    rh = jnp.maximum(h_ref[0], 0.0)
    rg = jnp.maximum(_unpack_rows(g_ref[0]), 0.0)
    h2 = (jnp.dot(rh, wa_ref[0], preferred_element_type=jnp.float32)
          + jnp.dot(rg, wb_ref[0], preferred_element_type=jnp.float32))
    h2_ref[0] = h2
    hp_ref[0] = _pack_rows(h2)
    a = a_ref[0]
    p_ref[0] = jnp.dot(h2, a[:D], preferred_element_type=jnp.float32)
    q_ref[0] = jnp.dot(h2, a[D:], preferred_element_type=jnp.float32)


def _proj2(h, g, wa, wb, a):
    return pl.pallas_call(
        _proj2_body,
        grid=(M, NPAD // TB),
        in_specs=[
            pl.BlockSpec((1, TB, D), lambda m, n: (m, n, 0)),
            pl.BlockSpec((1, TB, DP), lambda m, n: (m, n, 0)),
            pl.BlockSpec((1, D, D), lambda m, n: (m, 0, 0)),
            pl.BlockSpec((1, D, D), lambda m, n: (m, 0, 0)),
            pl.BlockSpec((1, 2 * D, 1), lambda m, n: (m, 0, 0)),
        ],
        out_specs=[
            pl.BlockSpec((1, TB, D), lambda m, n: (m, n, 0)),
            pl.BlockSpec((1, TB, DP), lambda m, n: (m, n, 0)),
            pl.BlockSpec((1, TB, 1), lambda m, n: (m, n, 0)),
            pl.BlockSpec((1, TB, 1), lambda m, n: (m, n, 0)),
        ],
        out_shape=[
            jax.ShapeDtypeStruct((M, NPAD, D), jnp.float32),
            jax.ShapeDtypeStruct((M, NPAD, DP), jnp.float32),
            jax.ShapeDtypeStruct((M, NPAD, 1), jnp.float32),
            jax.ShapeDtypeStruct((M, NPAD, 1), jnp.float32),
        ],
    )(h, g, wa, wb, a)


_sc_mesh = plsc.VectorSubcoreMesh(core_axis_name="c", subcore_axis_name="s")


@functools.partial(
    pl.kernel,
    out_type=jax.ShapeDtypeStruct((TT, DP), jnp.float32),
    mesh=_sc_mesh,
    compiler_params=pltpu.CompilerParams(needs_layout_passes=False),
    scratch_types=[
        pltpu.VMEM_SHARED((NSTG, DP), jnp.float32),
        pltpu.VMEM((NPAD,), jnp.float32),
        pltpu.VMEM((NPW + S,), jnp.float32),
        pltpu.VMEM((NPW * S,), jnp.int32),
        pltpu.VMEM((BLK * S, DP), jnp.float32),
        pltpu.VMEM((BLK * S, DP), jnp.float32),
        pltpu.VMEM((2 * BLK, DP), jnp.float32),
        pltpu.SemaphoreType.DMA,
        pltpu.SemaphoreType.DMA,
        pltpu.SemaphoreType.DMA,
    ],
)
def _sc_agg(h_hbm, nbr_hbm, p_hbm, q_hbm, out_hbm,
            shared, q_v, p_v, nbr_v, rows0_v, rows1_v, ob_v,
            sem0, sem1, semo):
    cid = lax.axis_index("c")
    sid = lax.axis_index("s")
    wid = sid * 2 + cid
    HN = BLK * S // 4

    def gather(blk, rows_v, sem):
        for k in range(4):
            idxk = nbr_v.at[pl.ds(blk * (BLK * S) + k * HN, HN)]
            pltpu.async_copy(shared.at[idxk],
                             rows_v.at[pl.ds(k * HN, HN)], sem)

    def wait_rows(rows_v, sem):
        for k in range(4):
            pltpu.make_async_copy(
                shared.at[nbr_v.at[pl.ds(0, HN)]],
                rows_v.at[pl.ds(k * HN, HN)], sem).wait()

    def compute(blk, rows_v, half):
        for i in range(BLK):
            t = blk * BLK + i
            idx = nbr_v[pl.ds(t * S, S)]
            qn = plsc.load_gather(q_v, [idx])
            pv = p_v[pl.ds(t, S)][0]
            z = qn + pv
            e = jnp.maximum(z, ALPHA * z)
            ex = jnp.exp(e - jnp.max(e))
            att = ex / jnp.sum(ex)
            w = [att[s] for s in range(S)]
            himask = jnp.full((16,), -65536, jnp.int32)
            rnd = jnp.full((16,), 0x8000, jnp.int32)
            for dblk in range(DP // 16):
                sl = pl.ds(dblk * 16, 16)
                acc_a = None
                acc_b = None
                for s in range(S):
                    iw = plsc.bitcast(rows_v[i * S + s, sl], jnp.int32)
                    va = plsc.bitcast(iw << 16, jnp.float32)
                    vb = plsc.bitcast(iw & himask, jnp.float32)
                    if acc_a is None:
                        acc_a = w[s] * va
                        acc_b = w[s] * vb
                    else:
                        acc_a = acc_a + w[s] * va
                        acc_b = acc_b + w[s] * vb
                ia = plsc.bitcast(acc_a, jnp.int32)
                ib = plsc.bitcast(acc_b, jnp.int32)
                lo16 = lax.shift_right_logical(ia + rnd, 16)
                hi16 = (ib + rnd) & himask
                ob_v[half * BLK + i, sl] = plsc.bitcast(lo16 | hi16,
                                                        jnp.float32)

    def phase(m, carry):
        tb = m * NPAD
        pbase = wid * NPW
        pltpu.sync_copy(h_hbm.at[pl.ds(tb + sid * STG, STG)],
                        shared.at[pl.ds(sid * STG, STG)])
        pltpu.sync_copy(q_hbm.at[pl.ds(tb, NPAD)], q_v)
        pltpu.sync_copy(p_hbm.at[pl.ds(tb + pbase, NPW)],
                        p_v.at[pl.ds(0, NPW)])
        pltpu.sync_copy(nbr_hbm.at[pl.ds((tb + pbase) * S, NPW * S)], nbr_v)
        plsc.subcore_barrier()

        obase = tb + pbase
        gather(0, rows0_v, sem0)

        def pair(j, _):
            b0 = 2 * j
            gather(b0 + 1, rows1_v, sem1)
            @pl.when(j > 0)
            def _drain():
                pltpu.make_async_copy(
                    ob_v, out_hbm.at[pl.ds(obase, 2 * BLK)], semo).wait()
            wait_rows(rows0_v, sem0)
            compute(b0, rows0_v, 0)

            @pl.when(j < NPAIR - 1)
            def _prefetch():
                gather(b0 + 2, rows0_v, sem0)
            wait_rows(rows1_v, sem1)
            compute(b0 + 1, rows1_v, 1)
            pltpu.async_copy(
                ob_v, out_hbm.at[pl.ds(obase + b0 * BLK, 2 * BLK)], semo)
            return _

        lax.fori_loop(0, NPAIR, pair, None)
        pltpu.make_async_copy(
            ob_v, out_hbm.at[pl.ds(obase, 2 * BLK)], semo).wait()
        plsc.subcore_barrier()
        return carry

    lax.fori_loop(0, M, phase, None)


def _final_body(h_ref, g_ref, amp_ref, wl_ref, b_ref, o_ref):
    amp = amp_ref[...]
    a_top, a_bot = amp[:D], amp[D:]
    hs, gs, es = [], [], []
    for m in range(M):
        hm = jnp.maximum(h_ref[m], 0.0)
        gm = jnp.maximum(_unpack_rows(g_ref[m]), 0.0)
        e = (jnp.dot(hm, a_top, preferred_element_type=jnp.float32)
             + jnp.dot(gm, a_bot, preferred_element_type=jnp.float32))
        e = jnp.maximum(e, ALPHA * e)
        hs.append(hm)
        gs.append(gm)
        es.append(e)
    mx = jnp.maximum(jnp.maximum(es[0], es[1]), es[2])
    ws = [jnp.exp(e - mx) for e in es]
    tot = ws[0] + ws[1] + ws[2]
    ph = (ws[0] * hs[0] + ws[1] * hs[1] + ws[2] * hs[2]) / tot
    pg = (ws[0] * gs[0] + ws[1] * gs[1] + ws[2] * gs[2]) / tot
    wl = wl_ref[...]
    logits = (jnp.dot(ph, wl[:D], preferred_element_type=jnp.float32)
              + jnp.dot(pg, wl[D:], preferred_element_type=jnp.float32)
              + b_ref[...])
    r = jnp.maximum(logits, 0.0)
    rmx = jnp.max(r, axis=1, keepdims=True)
    lse = jnp.log(jnp.sum(jnp.exp(r - rmx), axis=1, keepdims=True)) + rmx
    o_ref[...] = r - lse


def _final(h2, g2, a_mp, w_lin, b_lin):
    return pl.pallas_call(
        _final_body,
        grid=(NPAD // TB,),
        in_specs=[
            pl.BlockSpec((M, TB, D), lambda n: (0, n, 0)),
            pl.BlockSpec((M, TB, DP), lambda n: (0, n, 0)),
            pl.BlockSpec((2 * D, 1), lambda n: (0, 0)),
            pl.BlockSpec((2 * D, NCLASS), lambda n: (0, 0)),
            pl.BlockSpec((NCLASS,), lambda n: (0,)),
        ],
        out_specs=pl.BlockSpec((TB, NCLASS), lambda n: (n, 0)),
        out_shape=jax.ShapeDtypeStruct((NPAD, NCLASS), jnp.float32),
    )(h2, g2, a_mp, w_lin, b_lin)


def _flat_nbr(nbr):
    nf = jnp.pad(nbr, ((0, 0), (0, NPAD - N), (0, 0)))
    return nf.reshape(TT * S)


def kernel(x, nbr1, nbr2, W1, a1, W2, a2, a_mp, W_lin, b_lin):
    xp = jnp.pad(x, ((0, NPAD - N), (0, 0)))
    nbr1f = _flat_nbr(nbr1)
    nbr2f = _flat_nbr(nbr2)

    h1, h1p, p1, q1 = _proj(xp, W1, a1, 512)
    agg1 = _sc_agg(h1p.reshape(TT, DP), nbr1f,
                   p1.reshape(TT), q1.reshape(TT))
    h2, h2p, p2, q2 = _proj2(h1, agg1.reshape(M, NPAD, DP),
                             W2[:, :D, :], W2[:, D:, :], a2)
    agg2 = _sc_agg(h2p.reshape(TT, DP), nbr2f,
                   p2.reshape(TT), q2.reshape(TT))
    out = _final(h2, agg2.reshape(M, NPAD, DP), a_mp, W_lin, b_lin)
    return out[:N]
